# baseline (speedup 1.0000x reference)
import functools

import jax
import jax.numpy as jnp
from jax import lax
from jax.experimental import pallas as pl
from jax.experimental.pallas import tpu as pltpu
from jax.experimental.pallas import tpu_sc as plsc

E = 8
K = 2
N = 4096
H = 1024
F = 2048
BLK = 128
NPAD = N * K + E * BLK
NBLK = NPAD // BLK
CH = 1024

_INTERPRET = False


def _router_body(x_ref, gw_ref, gb_ref, dest_ref, emap_ref):
    xb = x_ref[...].astype(jnp.bfloat16)
    gw = gw_ref[...].astype(jnp.bfloat16)
    scores = lax.dot_general(xb, gw, (((1,), (1,)), ((), ())),
                             preferred_element_type=jnp.float32)
    scores = scores + gb_ref[...]

    iota_e = lax.broadcasted_iota(jnp.int32, (N, E), 1)
    m1 = jnp.max(scores, axis=1, keepdims=True)
    e0 = jnp.min(jnp.where(scores == m1, iota_e, E), axis=1, keepdims=True)
    s2 = jnp.where(iota_e == e0, -jnp.inf, scores)
    m2 = jnp.max(s2, axis=1, keepdims=True)
    e1 = jnp.min(jnp.where(s2 == m2, iota_e, E), axis=1, keepdims=True)

    mask0 = (iota_e == e0).astype(jnp.float32)
    mask1 = (iota_e == e1).astype(jnp.float32)
    counts = (jnp.sum(mask0, axis=0, keepdims=True)
              + jnp.sum(mask1, axis=0, keepdims=True))

    iota_1e = lax.broadcasted_iota(jnp.int32, (1, E), 1)

    def off_step(e, carry):
        off_vec, cur = carry
        off_vec = off_vec + cur * (iota_1e == e).astype(jnp.float32)
        cnt_e = jnp.sum(jnp.where(iota_1e == e, counts, 0.0))
        cur = jnp.ceil((cur + cnt_e) / BLK) * BLK
        return off_vec, cur

    off_vec, total = lax.fori_loop(0, E, off_step,
                                   (jnp.zeros((1, E), jnp.float32),
                                    jnp.float32(0.0)))

    nused = (total / BLK).astype(jnp.int32)
    bidx = lax.broadcasted_iota(jnp.int32, (128, 1), 0)
    bstart = (BLK * bidx).astype(jnp.float32)
    emap = jnp.sum((off_vec <= bstart).astype(jnp.int32), axis=1,
                   keepdims=True) - 1
    emap = jnp.clip(emap, 0, E - 1)
    em_last = jnp.sum(jnp.where(bidx == nused - 1, emap, 0))
    emap = jnp.where(bidx < nused, emap, em_last)
    emap = jnp.where(bidx == 127, nused, emap)
    emap_ref[...] = emap.reshape(1, 128)

    ii = lax.broadcasted_iota(jnp.int32, (CH, CH), 0)
    jj = lax.broadcasted_iota(jnp.int32, (CH, CH), 1)
    tri = (jj < ii).astype(jnp.float32)
    carry = jnp.zeros((1, E), jnp.float32)
    for c in range(N * K // CH):
        base = (c % (N // CH)) * CH
        m = mask0 if c < (N // CH) else mask1
        mc = lax.slice(m, (base, 0), (base + CH, E))
        prefix = lax.dot_general(tri, mc, (((1,), (0,)), ((), ())),
                                 precision=lax.Precision.HIGHEST,
                                 preferred_element_type=jnp.float32)
        dest_c = jnp.sum((prefix + carry + off_vec) * mc, axis=1)
        dest_ref[c // (N // CH), pl.ds(base, CH)] = dest_c.astype(jnp.int32)
        carry = carry + jnp.sum(mc, axis=0, keepdims=True)


def _router(x_flat, gate_W, gate_b):
    return pl.pallas_call(
        _router_body,
        out_shape=(jax.ShapeDtypeStruct((K, N), jnp.int32),
                   jax.ShapeDtypeStruct((1, 128), jnp.int32)),
        interpret=_INTERPRET,
    )(x_flat, gate_W, gate_b.reshape(1, E))


def _ffn_body(emap_ref, x_ref, w1_ref, b1_ref, w2_ref, b2_ref, y_ref):
    @pl.when(pl.program_id(0) < emap_ref[127])
    def _():
        xb = x_ref[...].astype(jnp.bfloat16)
        h = lax.dot_general(xb, w1_ref[0], (((1,), (1,)), ((), ())),
                            preferred_element_type=jnp.float32)
        h = h + b1_ref[0]
        h = 0.5 * h * (1.0 + lax.erf(h * 0.7071067811865476))
        y = lax.dot_general(h.astype(jnp.bfloat16), w2_ref[0],
                            (((1,), (1,)), ((), ())),
                            preferred_element_type=jnp.float32)
        y_ref[...] = y + b2_ref[0]


def _ffn(emap, x_pad, w1, b1, w2, b2):
    grid_spec = pltpu.PrefetchScalarGridSpec(
        num_scalar_prefetch=1,
        grid=(NBLK,),
        in_specs=[
            pl.BlockSpec((BLK, H), lambda b, em: (b, 0)),
            pl.BlockSpec((1, F, H), lambda b, em: (em[b], 0, 0)),
            pl.BlockSpec((1, 1, F), lambda b, em: (em[b], 0, 0)),
            pl.BlockSpec((1, H, F), lambda b, em: (em[b], 0, 0)),
            pl.BlockSpec((1, 1, H), lambda b, em: (em[b], 0, 0)),
        ],
        out_specs=pl.BlockSpec((BLK, H), lambda b, em: (b, 0)),
    )
    return pl.pallas_call(
        _ffn_body,
        grid_spec=grid_spec,
        out_shape=jax.ShapeDtypeStruct((NPAD, H), jnp.float32),
        interpret=_INTERPRET,
    )(emap, x_pad, w1, b1, w2, b2)


_NC, _NS = 2, 16
_NW = _NC * _NS
_SB = 64
_CB = 32


@functools.cache
def _make_scatter():
    mesh = plsc.VectorSubcoreMesh(core_axis_name="c", subcore_axis_name="s")

    @functools.partial(
        pl.kernel, mesh=mesh,
        out_type=jax.ShapeDtypeStruct((NPAD, H), jnp.float32),
        scratch_types=[
            pltpu.VMEM((_SB,), jnp.int32),
            pltpu.VMEM((_SB, H), jnp.float32),
            pltpu.SemaphoreType.DMA,
        ],
    )
    def scatter_k(x_hbm, dest_hbm, xpad_hbm, idx_v, rows_v, sem):
        wid = lax.axis_index("s") * _NC + lax.axis_index("c")
        ppw = N * K // _NW
        k = wid // (N // ppw)
        t_base = (wid % (N // ppw)) * ppw

        def batch(i, _):
            t0 = t_base + i * _SB
            pltpu.sync_copy(dest_hbm.at[k, pl.ds(t0, _SB)], idx_v)
            pltpu.sync_copy(x_hbm.at[pl.ds(t0, _SB), :], rows_v)
            pltpu.async_copy(rows_v, xpad_hbm.at[idx_v], sem).wait()
            return 0

        lax.fori_loop(0, ppw // _SB, batch, 0)

    return scatter_k


@functools.cache
def _make_combine():
    mesh = plsc.VectorSubcoreMesh(core_axis_name="c", subcore_axis_name="s")

    @functools.partial(
        pl.kernel, mesh=mesh,
        out_type=jax.ShapeDtypeStruct((N, H), jnp.float32),
        scratch_types=[
            pltpu.VMEM((_CB,), jnp.int32),
            pltpu.VMEM((_CB,), jnp.int32),
            pltpu.VMEM((_CB, H), jnp.float32),
            pltpu.VMEM((_CB, H), jnp.float32),
            pltpu.SemaphoreType.DMA,
            pltpu.SemaphoreType.DMA,
        ],
    )
    def combine_k(ypad_hbm, dest_hbm, out_hbm, idx0, idx1, buf0, buf1,
                  sem0, sem1):
        wid = lax.axis_index("s") * _NC + lax.axis_index("c")
        tpw = N // _NW

        def batch(i, _):
            t0 = wid * tpw + i * _CB
            pltpu.sync_copy(dest_hbm.at[0, pl.ds(t0, _CB)], idx0)
            pltpu.sync_copy(dest_hbm.at[1, pl.ds(t0, _CB)], idx1)
            cp0 = pltpu.async_copy(ypad_hbm.at[idx0], buf0, sem0)
            cp1 = pltpu.async_copy(ypad_hbm.at[idx1], buf1, sem1)
            cp0.wait()
            cp1.wait()

            def row(r, _):
                for c in range(H // 16):
                    buf0[r, pl.ds(16 * c, 16)] = (
                        buf0[r, pl.ds(16 * c, 16)]
                        + buf1[r, pl.ds(16 * c, 16)])
                return 0

            lax.fori_loop(0, _CB, row, 0)
            pltpu.sync_copy(buf0, out_hbm.at[pl.ds(t0, _CB), :])
            return 0

        lax.fori_loop(0, tpw // _CB, batch, 0)

    return combine_k


def kernel(x, gate_W, gate_b, fc1_W, fc1_b, fc2_W, fc2_b):
    B, S, _ = x.shape
    x_flat = x.reshape(B * S, H)
    dest, emap = _router(x_flat, gate_W, gate_b)
    emap = emap.reshape(128)

    x_pad = _make_scatter()(x_flat, dest)

    y_pad = _ffn(emap, x_pad,
                 fc1_W.astype(jnp.bfloat16), fc1_b.reshape(E, 1, F),
                 fc2_W.astype(jnp.bfloat16), fc2_b.reshape(E, 1, H))

    out = _make_combine()(y_pad, dest)
    return out.reshape(B, S, H)


# speedup vs baseline: 1.3565x; 1.3565x over previous
import functools

import jax
import jax.numpy as jnp
from jax import lax
from jax.experimental import pallas as pl
from jax.experimental.pallas import tpu as pltpu
from jax.experimental.pallas import tpu_sc as plsc

E = 8
K = 2
N = 4096
H = 1024
F = 2048
BLK = 256
NPAD = N * K + E * BLK
NBLK = NPAD // BLK
CH = 1024

_INTERPRET = False


def _router_body(x_ref, gw_ref, gb_ref, dest_ref, emap_ref):
    xb = x_ref[...].astype(jnp.bfloat16)
    gw = gw_ref[...].astype(jnp.bfloat16)
    scores = lax.dot_general(xb, gw, (((1,), (1,)), ((), ())),
                             preferred_element_type=jnp.float32)
    scores = scores + gb_ref[...]

    iota_e = lax.broadcasted_iota(jnp.int32, (N, E), 1)
    m1 = jnp.max(scores, axis=1, keepdims=True)
    e0 = jnp.min(jnp.where(scores == m1, iota_e, E), axis=1, keepdims=True)
    s2 = jnp.where(iota_e == e0, -jnp.inf, scores)
    m2 = jnp.max(s2, axis=1, keepdims=True)
    e1 = jnp.min(jnp.where(s2 == m2, iota_e, E), axis=1, keepdims=True)

    mask0 = (iota_e == e0).astype(jnp.float32)
    mask1 = (iota_e == e1).astype(jnp.float32)
    counts = (jnp.sum(mask0, axis=0, keepdims=True)
              + jnp.sum(mask1, axis=0, keepdims=True))

    iota_1e = lax.broadcasted_iota(jnp.int32, (1, E), 1)

    def off_step(e, carry):
        off_vec, cur = carry
        off_vec = off_vec + cur * (iota_1e == e).astype(jnp.float32)
        cnt_e = jnp.sum(jnp.where(iota_1e == e, counts, 0.0))
        cur = jnp.ceil((cur + cnt_e) / BLK) * BLK
        return off_vec, cur

    off_vec, total = lax.fori_loop(0, E, off_step,
                                   (jnp.zeros((1, E), jnp.float32),
                                    jnp.float32(0.0)))

    nused = (total / BLK).astype(jnp.int32)
    bidx = lax.broadcasted_iota(jnp.int32, (128, 1), 0)
    bstart = (BLK * bidx).astype(jnp.float32)
    emap = jnp.sum((off_vec <= bstart).astype(jnp.int32), axis=1,
                   keepdims=True) - 1
    emap = jnp.clip(emap, 0, E - 1)
    em_last = jnp.sum(jnp.where(bidx == nused - 1, emap, 0))
    emap = jnp.where(bidx < nused, emap, em_last)
    emap = jnp.where(bidx == 127, nused, emap)
    emap_ref[...] = emap.reshape(1, 128)

    ii = lax.broadcasted_iota(jnp.int32, (CH, CH), 0)
    jj = lax.broadcasted_iota(jnp.int32, (CH, CH), 1)
    tri = (jj < ii).astype(jnp.float32)
    carry = jnp.zeros((1, E), jnp.float32)
    for c in range(N * K // CH):
        base = (c % (N // CH)) * CH
        m = mask0 if c < (N // CH) else mask1
        mc = lax.slice(m, (base, 0), (base + CH, E))
        prefix = lax.dot_general(tri, mc, (((1,), (0,)), ((), ())),
                                 precision=lax.Precision.HIGHEST,
                                 preferred_element_type=jnp.float32)
        dest_c = jnp.sum((prefix + carry + off_vec) * mc, axis=1)
        dest_ref[c // (N // CH), pl.ds(base, CH)] = dest_c.astype(jnp.int32)
        carry = carry + jnp.sum(mc, axis=0, keepdims=True)


def _router(x_flat, gate_W, gate_b):
    return pl.pallas_call(
        _router_body,
        out_shape=(jax.ShapeDtypeStruct((K, N), jnp.int32),
                   jax.ShapeDtypeStruct((1, 128), jnp.int32)),
        interpret=_INTERPRET,
    )(x_flat, gate_W, gate_b.reshape(1, E))


def _ffn_body(emap_ref, x_ref, w1_ref, b1_ref, w2_ref, b2_ref, y_ref):
    @pl.when(pl.program_id(0) < emap_ref[127])
    def _():
        xb = x_ref[...].astype(jnp.bfloat16)
        h = lax.dot_general(xb, w1_ref[0], (((1,), (1,)), ((), ())),
                            preferred_element_type=jnp.float32)
        h = h + b1_ref[0]
        h = 0.5 * h * (1.0 + lax.erf(h * 0.7071067811865476))
        y = lax.dot_general(h.astype(jnp.bfloat16), w2_ref[0],
                            (((1,), (1,)), ((), ())),
                            preferred_element_type=jnp.float32)
        y_ref[...] = y + b2_ref[0]


def _ffn(emap, x_pad, w1, b1, w2, b2):
    grid_spec = pltpu.PrefetchScalarGridSpec(
        num_scalar_prefetch=1,
        grid=(NBLK,),
        in_specs=[
            pl.BlockSpec((BLK, H), lambda b, em: (b, 0)),
            pl.BlockSpec((1, F, H), lambda b, em: (em[b], 0, 0)),
            pl.BlockSpec((1, 1, F), lambda b, em: (em[b], 0, 0)),
            pl.BlockSpec((1, H, F), lambda b, em: (em[b], 0, 0)),
            pl.BlockSpec((1, 1, H), lambda b, em: (em[b], 0, 0)),
        ],
        out_specs=pl.BlockSpec((BLK, H), lambda b, em: (b, 0)),
    )
    return pl.pallas_call(
        _ffn_body,
        grid_spec=grid_spec,
        out_shape=jax.ShapeDtypeStruct((NPAD, H), jnp.float32),
        interpret=_INTERPRET,
    )(emap, x_pad, w1, b1, w2, b2)


_NC, _NS = 2, 16
_NW = _NC * _NS
_SB = 64
_CB = 32


@functools.cache
def _make_scatter():
    mesh = plsc.VectorSubcoreMesh(core_axis_name="c", subcore_axis_name="s")

    @functools.partial(
        pl.kernel, mesh=mesh,
        out_type=jax.ShapeDtypeStruct((NPAD, H), jnp.float32),
        scratch_types=[
            pltpu.VMEM((_SB,), jnp.int32),
            pltpu.VMEM((_SB, H), jnp.float32),
            pltpu.SemaphoreType.DMA,
        ],
    )
    def scatter_k(x_hbm, dest_hbm, xpad_hbm, idx_v, rows_v, sem):
        wid = lax.axis_index("s") * _NC + lax.axis_index("c")
        ppw = N * K // _NW
        k = wid // (N // ppw)
        t_base = (wid % (N // ppw)) * ppw

        def batch(i, _):
            t0 = t_base + i * _SB
            pltpu.sync_copy(dest_hbm.at[k, pl.ds(t0, _SB)], idx_v)
            pltpu.sync_copy(x_hbm.at[pl.ds(t0, _SB), :], rows_v)
            pltpu.async_copy(rows_v, xpad_hbm.at[idx_v], sem).wait()
            return 0

        lax.fori_loop(0, ppw // _SB, batch, 0)

    return scatter_k


@functools.cache
def _make_combine():
    mesh = plsc.VectorSubcoreMesh(core_axis_name="c", subcore_axis_name="s")

    @functools.partial(
        pl.kernel, mesh=mesh,
        out_type=jax.ShapeDtypeStruct((N, H), jnp.float32),
        scratch_types=[
            pltpu.VMEM((_CB,), jnp.int32),
            pltpu.VMEM((_CB,), jnp.int32),
            pltpu.VMEM((_CB, H), jnp.float32),
            pltpu.VMEM((_CB, H), jnp.float32),
            pltpu.SemaphoreType.DMA,
            pltpu.SemaphoreType.DMA,
        ],
    )
    def combine_k(ypad_hbm, dest_hbm, out_hbm, idx0, idx1, buf0, buf1,
                  sem0, sem1):
        wid = lax.axis_index("s") * _NC + lax.axis_index("c")
        tpw = N // _NW

        def batch(i, _):
            t0 = wid * tpw + i * _CB
            pltpu.sync_copy(dest_hbm.at[0, pl.ds(t0, _CB)], idx0)
            pltpu.sync_copy(dest_hbm.at[1, pl.ds(t0, _CB)], idx1)
            cp0 = pltpu.async_copy(ypad_hbm.at[idx0], buf0, sem0)
            cp1 = pltpu.async_copy(ypad_hbm.at[idx1], buf1, sem1)
            cp0.wait()
            cp1.wait()

            def row(r, _):
                for c in range(H // 16):
                    buf0[r, pl.ds(16 * c, 16)] = (
                        buf0[r, pl.ds(16 * c, 16)]
                        + buf1[r, pl.ds(16 * c, 16)])
                return 0

            lax.fori_loop(0, _CB, row, 0)
            pltpu.sync_copy(buf0, out_hbm.at[pl.ds(t0, _CB), :])
            return 0

        lax.fori_loop(0, tpw // _CB, batch, 0)

    return combine_k


def kernel(x, gate_W, gate_b, fc1_W, fc1_b, fc2_W, fc2_b):
    B, S, _ = x.shape
    x_flat = x.reshape(B * S, H)
    dest, emap = _router(x_flat, gate_W, gate_b)
    emap = emap.reshape(128)

    x_pad = _make_scatter()(x_flat, dest)

    y_pad = _ffn(emap, x_pad,
                 fc1_W.astype(jnp.bfloat16), fc1_b.reshape(E, 1, F),
                 fc2_W.astype(jnp.bfloat16), fc2_b.reshape(E, 1, H))

    out = _make_combine()(y_pad, dest)
    return out.reshape(B, S, H)


# speedup vs baseline: 1.3888x; 1.0238x over previous
import functools

import jax
import jax.numpy as jnp
from jax import lax
from jax.experimental import pallas as pl
from jax.experimental.pallas import tpu as pltpu
from jax.experimental.pallas import tpu_sc as plsc

E = 8
K = 2
N = 4096
H = 1024
F = 2048
BLK = 256
NPAD = N * K + E * BLK
NBLK = NPAD // BLK
CH = 1024

_INTERPRET = False


def _router_body(x_ref, gw_ref, gb_ref, dest_ref, emap_ref):
    xb = x_ref[...].astype(jnp.bfloat16)
    gw = gw_ref[...].astype(jnp.bfloat16)
    scores = lax.dot_general(xb, gw, (((1,), (1,)), ((), ())),
                             preferred_element_type=jnp.float32)
    scores = scores + gb_ref[...]

    iota_e = lax.broadcasted_iota(jnp.int32, (N, E), 1)
    m1 = jnp.max(scores, axis=1, keepdims=True)
    e0 = jnp.min(jnp.where(scores == m1, iota_e, E), axis=1, keepdims=True)
    s2 = jnp.where(iota_e == e0, -jnp.inf, scores)
    m2 = jnp.max(s2, axis=1, keepdims=True)
    e1 = jnp.min(jnp.where(s2 == m2, iota_e, E), axis=1, keepdims=True)

    mask0 = (iota_e == e0).astype(jnp.float32)
    mask1 = (iota_e == e1).astype(jnp.float32)
    counts = (jnp.sum(mask0, axis=0, keepdims=True)
              + jnp.sum(mask1, axis=0, keepdims=True))

    iota_1e = lax.broadcasted_iota(jnp.int32, (1, E), 1)

    def off_step(e, carry):
        off_vec, cur = carry
        off_vec = off_vec + cur * (iota_1e == e).astype(jnp.float32)
        cnt_e = jnp.sum(jnp.where(iota_1e == e, counts, 0.0))
        cur = jnp.ceil((cur + cnt_e) / BLK) * BLK
        return off_vec, cur

    off_vec, total = lax.fori_loop(0, E, off_step,
                                   (jnp.zeros((1, E), jnp.float32),
                                    jnp.float32(0.0)))

    nused = (total / BLK).astype(jnp.int32)
    bidx = lax.broadcasted_iota(jnp.int32, (128, 1), 0)
    bstart = (BLK * bidx).astype(jnp.float32)
    emap = jnp.sum((off_vec <= bstart).astype(jnp.int32), axis=1,
                   keepdims=True) - 1
    emap = jnp.clip(emap, 0, E - 1)
    em_last = jnp.sum(jnp.where(bidx == nused - 1, emap, 0))
    emap = jnp.where(bidx < nused, emap, em_last)
    emap = jnp.where(bidx == 127, nused, emap)
    emap_ref[...] = emap.reshape(1, 128)

    ii = lax.broadcasted_iota(jnp.int32, (CH, CH), 0)
    jj = lax.broadcasted_iota(jnp.int32, (CH, CH), 1)
    tri = (jj < ii).astype(jnp.float32)
    carry = jnp.zeros((1, E), jnp.float32)
    for c in range(N * K // CH):
        base = (c % (N // CH)) * CH
        m = mask0 if c < (N // CH) else mask1
        mc = lax.slice(m, (base, 0), (base + CH, E))
        prefix = lax.dot_general(tri, mc, (((1,), (0,)), ((), ())),
                                 precision=lax.Precision.HIGHEST,
                                 preferred_element_type=jnp.float32)
        dest_c = jnp.sum((prefix + carry + off_vec) * mc, axis=1)
        dest_ref[c // (N // CH), pl.ds(base, CH)] = dest_c.astype(jnp.int32)
        carry = carry + jnp.sum(mc, axis=0, keepdims=True)


def _router(x_flat, gate_W, gate_b):
    return pl.pallas_call(
        _router_body,
        out_shape=(jax.ShapeDtypeStruct((K, N), jnp.int32),
                   jax.ShapeDtypeStruct((1, 128), jnp.int32)),
        interpret=_INTERPRET,
    )(x_flat, gate_W, gate_b.reshape(1, E))


def _ffn_body(emap_ref, x_ref, w1_ref, b1_ref, w2_ref, b2_ref, y_ref):
    @pl.when(pl.program_id(0) < emap_ref[127])
    def _():
        xb = x_ref[...].astype(jnp.bfloat16)
        h = lax.dot_general(xb, w1_ref[0], (((1,), (1,)), ((), ())),
                            preferred_element_type=jnp.float32)
        h = h + b1_ref[0]
        h = 0.5 * h * (1.0 + lax.erf(h * 0.7071067811865476))
        y = lax.dot_general(h.astype(jnp.bfloat16), w2_ref[0],
                            (((1,), (1,)), ((), ())),
                            preferred_element_type=jnp.float32)
        y_ref[...] = y + b2_ref[0]


def _ffn(emap, x_pad, w1, b1, w2, b2):
    grid_spec = pltpu.PrefetchScalarGridSpec(
        num_scalar_prefetch=1,
        grid=(NBLK,),
        in_specs=[
            pl.BlockSpec((BLK, H), lambda b, em: (b, 0)),
            pl.BlockSpec((1, F, H), lambda b, em: (em[b], 0, 0)),
            pl.BlockSpec((1, 1, F), lambda b, em: (em[b], 0, 0)),
            pl.BlockSpec((1, H, F), lambda b, em: (em[b], 0, 0)),
            pl.BlockSpec((1, 1, H), lambda b, em: (em[b], 0, 0)),
        ],
        out_specs=pl.BlockSpec((BLK, H), lambda b, em: (b, 0)),
    )
    return pl.pallas_call(
        _ffn_body,
        grid_spec=grid_spec,
        out_shape=jax.ShapeDtypeStruct((NPAD, H), jnp.float32),
        interpret=_INTERPRET,
    )(emap, x_pad, w1, b1, w2, b2)


_NC, _NS = 2, 16
_NW = _NC * _NS
_SB = 32
_CB = 16


@functools.cache
def _make_scatter():
    mesh = plsc.VectorSubcoreMesh(core_axis_name="c", subcore_axis_name="s")
    nb = N * K // _NW // _SB

    @functools.partial(
        pl.kernel, mesh=mesh,
        out_type=jax.ShapeDtypeStruct((NPAD, H), jnp.float32),
        scratch_types=[
            pltpu.VMEM((nb, _SB), jnp.int32),
            pltpu.VMEM((_SB, H), jnp.float32),
            pltpu.VMEM((_SB, H), jnp.float32),
            pltpu.SemaphoreType.DMA,
            pltpu.SemaphoreType.DMA,
            pltpu.SemaphoreType.DMA,
            pltpu.SemaphoreType.DMA,
        ],
    )
    def scatter_k(x_hbm, dest_hbm, xpad_hbm, idx_v, rows0, rows1,
                  sl0, sl1, ss0, ss1):
        wid = lax.axis_index("s") * _NC + lax.axis_index("c")
        nw_per_k = _NW // K
        k = wid // nw_per_k
        t_base = (wid % nw_per_k) * (nb * _SB)
        pltpu.sync_copy(dest_hbm.at[k, wid % nw_per_k], idx_v)

        rows = (rows0, rows1)
        semL = (sl0, sl1)
        semS = (ss0, ss1)
        loads = {}
        scats = {}
        loads[0] = pltpu.async_copy(
            x_hbm.at[pl.ds(t_base, _SB), :], rows[0], semL[0])
        for i in range(nb):
            j = i % 2
            loads[i].wait()
            scats[i] = pltpu.async_copy(
                rows[j], xpad_hbm.at[idx_v.at[i]], semS[j])
            if i + 1 < nb:
                if i - 1 >= 0:
                    scats[i - 1].wait()
                loads[i + 1] = pltpu.async_copy(
                    x_hbm.at[pl.ds(t_base + (i + 1) * _SB, _SB), :],
                    rows[1 - j], semL[1 - j])
        scats[nb - 2].wait()
        scats[nb - 1].wait()

    return scatter_k


@functools.cache
def _make_combine():
    mesh = plsc.VectorSubcoreMesh(core_axis_name="c", subcore_axis_name="s")
    nb = N // _NW // _CB

    @functools.partial(
        pl.kernel, mesh=mesh,
        out_type=jax.ShapeDtypeStruct((N, H), jnp.float32),
        scratch_types=[
            pltpu.VMEM((nb, _CB), jnp.int32),
            pltpu.VMEM((nb, _CB), jnp.int32),
            pltpu.VMEM((_CB, H), jnp.float32),
            pltpu.VMEM((_CB, H), jnp.float32),
            pltpu.VMEM((_CB, H), jnp.float32),
            pltpu.VMEM((_CB, H), jnp.float32),
            pltpu.SemaphoreType.DMA,
            pltpu.SemaphoreType.DMA,
            pltpu.SemaphoreType.DMA,
            pltpu.SemaphoreType.DMA,
            pltpu.SemaphoreType.DMA,
            pltpu.SemaphoreType.DMA,
        ],
    )
    def combine_k(ypad_hbm, dest_hbm, out_hbm, idx0, idx1,
                  b0a, b0b, b1a, b1b, g0a, g0b, g1a, g1b, soa, sob):
        wid = lax.axis_index("s") * _NC + lax.axis_index("c")
        tpw = nb * _CB
        pltpu.sync_copy(dest_hbm.at[0, wid], idx0)
        pltpu.sync_copy(dest_hbm.at[1, wid], idx1)

        buf0 = (b0a, b0b)
        buf1 = (b1a, b1b)
        semG0 = (g0a, g0b)
        semG1 = (g1a, g1b)
        semO = (soa, sob)
        gath = {}
        stores = {}
        gath[0] = (pltpu.async_copy(ypad_hbm.at[idx0.at[0]], buf0[0], semG0[0]),
                   pltpu.async_copy(ypad_hbm.at[idx1.at[0]], buf1[0], semG1[0]))
        for i in range(nb):
            j = i % 2
            if i + 1 < nb:
                if i - 1 >= 0:
                    stores[i - 1].wait()
                gath[i + 1] = (
                    pltpu.async_copy(ypad_hbm.at[idx0.at[i + 1]],
                                     buf0[1 - j], semG0[1 - j]),
                    pltpu.async_copy(ypad_hbm.at[idx1.at[i + 1]],
                                     buf1[1 - j], semG1[1 - j]))
            gath[i][0].wait()
            gath[i][1].wait()

            def row(r, _, j=j):
                for c in range(H // 16):
                    buf0[j][r, pl.ds(16 * c, 16)] = (
                        buf0[j][r, pl.ds(16 * c, 16)]
                        + buf1[j][r, pl.ds(16 * c, 16)])
                return 0

            lax.fori_loop(0, _CB, row, 0)
            stores[i] = pltpu.async_copy(
                buf0[j], out_hbm.at[pl.ds(wid * tpw + i * _CB, _CB), :],
                semO[j])
        stores[nb - 2].wait()
        stores[nb - 1].wait()

    return combine_k


def kernel(x, gate_W, gate_b, fc1_W, fc1_b, fc2_W, fc2_b):
    B, S, _ = x.shape
    x_flat = x.reshape(B * S, H)
    dest, emap = _router(x_flat, gate_W, gate_b)
    emap = emap.reshape(128)

    x_pad = _make_scatter()(x_flat, dest.reshape(K, _NW // K, N * K // _NW // _SB, _SB))

    y_pad = _ffn(emap, x_pad,
                 fc1_W.astype(jnp.bfloat16), fc1_b.reshape(E, 1, F),
                 fc2_W.astype(jnp.bfloat16), fc2_b.reshape(E, 1, H))

    out = _make_combine()(y_pad, dest.reshape(K, _NW, N // _NW // _CB, _CB))
    return out.reshape(B, S, H)
